# baseline (speedup 1.0000x reference)
import functools
import math

import jax
import jax.numpy as jnp
from jax import lax
from jax.experimental import pallas as pl
from jax.experimental.pallas import tpu as pltpu
from jax.experimental.pallas import tpu_sc as plsc

B = 1
S = 2048
D = 2048
H = 16
HD = D // H
U = 38
UP = 40
INV_SQRT_D = 1.0 / math.sqrt(HD)
QBLK = 512
SBLK = 256


def _proj_kernel(q_ref, k_ref, wq_ref, wk_ref, bq_ref, bk_ref,
                 oq_ref, ok_ref):
    dn = (((1,), (1,)), ((), ()))
    oq_ref[...] = (
        jax.lax.dot_general(q_ref[...], wq_ref[...], dn,
                            preferred_element_type=jnp.float32)
        + bq_ref[...]
    )
    ok_ref[...] = (
        jax.lax.dot_general(k_ref[...], wk_ref[...], dn,
                            preferred_element_type=jnp.float32)
        + bk_ref[...]
    )


def _project_qk(q2d, k2d, Wq, bq, Wk, bk):
    return pl.pallas_call(
        _proj_kernel,
        grid=(S // SBLK,),
        in_specs=[
            pl.BlockSpec((SBLK, D), lambda i: (i, 0)),
            pl.BlockSpec((SBLK, D), lambda i: (i, 0)),
            pl.BlockSpec((D, D), lambda i: (0, 0)),
            pl.BlockSpec((D, D), lambda i: (0, 0)),
            pl.BlockSpec((1, D), lambda i: (0, 0)),
            pl.BlockSpec((1, D), lambda i: (0, 0)),
        ],
        out_specs=[
            pl.BlockSpec((SBLK, D), lambda i: (i, 0)),
            pl.BlockSpec((SBLK, D), lambda i: (i, 0)),
        ],
        out_shape=[
            jax.ShapeDtypeStruct((S, D), jnp.float32),
            jax.ShapeDtypeStruct((S, D), jnp.float32),
        ],
    )(q2d, k2d, Wq, Wk, bq.reshape(1, D), bk.reshape(1, D))


def _imp_kernel(qh_ref, kh_ref, imp_ref):
    k = kh_ref[...]
    dn = (((1,), (1,)), ((), ()))
    for j in range(S // QBLK):
        qblk = qh_ref[j * QBLK:(j + 1) * QBLK, :]
        s = jax.lax.dot_general(qblk, k, dn,
                                preferred_element_type=jnp.float32)
        s = s * INV_SQRT_D
        imp = jnp.max(s, axis=1) - jnp.mean(s, axis=1)
        imp_ref[0, 0, j * QBLK:(j + 1) * QBLK] = imp


def _importance(Q2d, K2d):
    out = pl.pallas_call(
        _imp_kernel,
        grid=(H,),
        in_specs=[
            pl.BlockSpec((S, HD), lambda h: (0, h)),
            pl.BlockSpec((S, HD), lambda h: (0, h)),
        ],
        out_specs=pl.BlockSpec((1, 1, S), lambda h: (h, 0, 0)),
        out_shape=jax.ShapeDtypeStruct((H, 1, S), jnp.float32),
    )(Q2d, K2d)
    return out.reshape(H, S)


L = 16
NVEC = S // L
NGRP = NVEC // L
UPW = 128
_NEG = float("-inf")
_BIG = 1 << 20


def _perm(x, sh):
    idx = lax.iota(jnp.int32, L) ^ sh
    return x.at[idx].get(mode="promise_in_bounds")


def _bmax(x):
    for sh in (1, 2, 4, 8):
        x = jnp.maximum(x, _perm(x, sh))
    return x


def _bmin(x):
    for sh in (1, 2, 4, 8):
        x = jnp.minimum(x, _perm(x, sh))
    return x


def _topk_sc_body(imp_hbm, idx_hbm, imp_v, idx_v):
    cid = lax.axis_index("c")
    sid = lax.axis_index("s")
    wid = sid * 2 + cid

    @pl.when(wid < H)
    def _():
        pltpu.sync_copy(imp_hbm.at[wid], imp_v)
        lanes = lax.iota(jnp.int32, L)
        negv = jnp.full((L,), _NEG, jnp.float32)
        bigv = jnp.full((L,), _BIG, jnp.int32)

        def setup_q(q):
            def b(t, acc):
                t2 = t * 2
                v0 = imp_v[pl.ds((q * L + t2) * L, L)]
                v1 = imp_v[pl.ds((q * L + t2 + 1) * L, L)]
                acc = jnp.where(lanes == t2, _bmax(v0), acc)
                return jnp.where(lanes == t2 + 1, _bmax(v1), acc)
            return lax.fori_loop(0, L // 2, b, negv)

        groups = [setup_q(q) for q in range(NGRP)]

        def round_body(u, carry):
            g = list(carry[:NGRP])
            acc = list(carry[NGRP:])
            m = g[0]
            for q in range(1, NGRP):
                m = jnp.maximum(m, g[q])
            gmv = _bmax(m)
            cand = bigv
            for q in range(NGRP):
                cand = jnp.minimum(
                    cand, jnp.where(g[q] == gmv, q * L + lanes, bigv))
            cs = _bmin(cand)[0]
            v = imp_v[pl.ds(cs * L, L)]
            ls = _bmin(jnp.where(v == gmv, lanes, bigv))[0]
            gidx = cs * L + ls
            for a in range(3):
                acc[a] = jnp.where(lanes == (u - a * L), gidx, acc[a])
            nv = jnp.where(lanes == ls, _NEG, v)
            imp_v[pl.ds(cs * L, L)] = nv
            nm = _bmax(nv)
            qs = cs >> 4
            ts = cs & (L - 1)
            qv = (lanes * 0) + qs
            for q in range(NGRP):
                g[q] = jnp.where((lanes + (qv - q) * 64) == ts, nm, g[q])
            return (*g, *acc)

        init = (*groups,
                jnp.full((L,), -1, jnp.int32),
                jnp.full((L,), -1, jnp.int32),
                jnp.full((L,), -1, jnp.int32))
        fin = lax.fori_loop(0, U, round_body, init)
        accs = fin[NGRP:]
        for a in range(UPW // L):
            idx_v[pl.ds(a * L, L)] = (accs[a] if a < 3 else
                                      jnp.full((L,), -1, jnp.int32))
        pltpu.sync_copy(idx_v, idx_hbm.at[wid])


def _topk(imp):
    f = pl.kernel(
        _topk_sc_body,
        out_type=jax.ShapeDtypeStruct((H, UPW), jnp.int32),
        mesh=plsc.VectorSubcoreMesh(core_axis_name="c", subcore_axis_name="s"),
        scratch_types=[
            pltpu.VMEM((S,), jnp.float32),
            pltpu.VMEM((UPW,), jnp.int32),
        ],
    )
    return f(imp)[:, :UP]


def _selattn_kernel(qh_ref, kh_ref, idx_ref, v_ref, wv_ref, bv_ref,
                    wo_ref, bo_ref, corr_ref, base_ref, vmean_ref):
    h = pl.program_id(0)

    @pl.when(h == 0)
    def _():
        vmean_ref[...] = (jnp.sum(v_ref[...].astype(jnp.float32), axis=0,
                                  keepdims=True) * (1.0 / S))
        base_ref[...] = bo_ref[...]

    dn = (((1,), (1,)), ((), ()))
    q = qh_ref[...]
    k = kh_ref[...]
    idx = idx_ref[0]
    oh = (jax.lax.broadcasted_iota(jnp.int32, (UP, S), 1) == idx)
    oh = oh.astype(jnp.float32)
    qsel = jnp.dot(oh, q, preferred_element_type=jnp.float32)
    s = jax.lax.dot_general(qsel, k, dn,
                            preferred_element_type=jnp.float32) * INV_SQRT_D
    m = jnp.max(s, axis=1, keepdims=True)
    e = jnp.exp(s - m)
    denom = jnp.sum(e, axis=1, keepdims=True) + 1e-8
    w = e / denom
    wv = jnp.dot(w.astype(jnp.bfloat16), v_ref[...],
                 preferred_element_type=jnp.float32)
    osel = jax.lax.dot_general(wv, wv_ref[...], dn,
                               preferred_element_type=jnp.float32)
    wsum = jnp.sum(w, axis=1, keepdims=True)
    osel = osel + wsum * bv_ref[0]
    dflt = (jax.lax.dot_general(vmean_ref[...], wv_ref[...], dn,
                                preferred_element_type=jnp.float32)
            + bv_ref[0])
    corr_ref[0] = osel - dflt
    base_ref[...] += jax.lax.dot_general(dflt, wo_ref[...], dn,
                                         preferred_element_type=jnp.float32)


def _selected_attention(Q2d, K2d, idx, v2d, Wv, bv, Wo, bo):
    idx3 = idx.reshape(H, UP, 1)
    bv3 = bv.reshape(H, 1, HD)
    return pl.pallas_call(
        _selattn_kernel,
        grid=(H,),
        in_specs=[
            pl.BlockSpec((S, HD), lambda h: (0, h)),
            pl.BlockSpec((S, HD), lambda h: (0, h)),
            pl.BlockSpec((1, UP, 1), lambda h: (h, 0, 0)),
            pl.BlockSpec((S, D), lambda h: (0, 0)),
            pl.BlockSpec((HD, D), lambda h: (h, 0)),
            pl.BlockSpec((1, 1, HD), lambda h: (h, 0, 0)),
            pl.BlockSpec((D, HD), lambda h: (0, h)),
            pl.BlockSpec((1, D), lambda h: (0, 0)),
        ],
        out_specs=[
            pl.BlockSpec((1, UP, HD), lambda h: (h, 0, 0)),
            pl.BlockSpec((1, D), lambda h: (0, 0)),
        ],
        out_shape=[
            jax.ShapeDtypeStruct((H, UP, HD), jnp.float32),
            jax.ShapeDtypeStruct((1, D), jnp.float32),
        ],
        scratch_shapes=[pltpu.VMEM((1, D), jnp.float32)],
        compiler_params=pltpu.CompilerParams(
            dimension_semantics=("arbitrary",)),
    )(Q2d, K2d, idx3, v2d.astype(jnp.bfloat16), Wv, bv3, Wo,
      bo.reshape(1, D))


def _assemble_kernel(idx_sref, corr_ref, base_ref, wo_ref, out_ref):
    h = pl.program_id(0)
    dn = (((1,), (1,)), ((), ()))

    @pl.when(h == 0)
    def _():
        out_ref[...] = jnp.broadcast_to(base_ref[...], (S, D))

    corr_out = jax.lax.dot_general(corr_ref[0], wo_ref[...], dn,
                                   preferred_element_type=jnp.float32)
    for i in range(U):
        r = idx_sref[h * UP + i]
        out_ref[pl.ds(r, 1), :] += corr_out[i:i + 1, :]


def _assemble(idx, corr, base, Wo):
    idx_flat = idx.reshape(H * UP)
    grid_spec = pltpu.PrefetchScalarGridSpec(
        num_scalar_prefetch=1,
        grid=(H,),
        in_specs=[
            pl.BlockSpec((1, UP, HD), lambda h, sref: (h, 0, 0)),
            pl.BlockSpec((1, D), lambda h, sref: (0, 0)),
            pl.BlockSpec((D, HD), lambda h, sref: (0, h)),
        ],
        out_specs=pl.BlockSpec((S, D), lambda h, sref: (0, 0)),
    )
    return pl.pallas_call(
        _assemble_kernel,
        grid_spec=grid_spec,
        out_shape=jax.ShapeDtypeStruct((S, D), jnp.float32),
        compiler_params=pltpu.CompilerParams(
            dimension_semantics=("arbitrary",)),
    )(idx_flat, corr, base, Wo)


@jax.jit
def kernel(queries, keys, values, Wq, bq, Wk, bk, Wv, bv, Wo, bo):
    q2d = queries.reshape(S, D)
    k2d = keys.reshape(S, D)
    v2d = values.reshape(S, D)
    Q2d, K2d = _project_qk(q2d, k2d, Wq, bq, Wk, bk)
    imp = _importance(Q2d, K2d)
    idx = _topk(imp)
    corr, base = _selected_attention(Q2d, K2d, idx, v2d, Wv, bv, Wo, bo)
    out = _assemble(idx, corr, base, Wo)
    return out.reshape(B, S, D)


# speedup vs baseline: 1.0640x; 1.0640x over previous
import functools
import math

import jax
import jax.numpy as jnp
from jax import lax
from jax.experimental import pallas as pl
from jax.experimental.pallas import tpu as pltpu
from jax.experimental.pallas import tpu_sc as plsc

B = 1
S = 2048
D = 2048
H = 16
HD = D // H
U = 38
UP = 40
INV_SQRT_D = 1.0 / math.sqrt(HD)
QBLK = 512
SBLK = 256


def _proj_kernel(q_ref, k_ref, wq_ref, wk_ref, bq_ref, bk_ref,
                 oq_ref, ok_ref):
    dn = (((1,), (1,)), ((), ()))
    oq_ref[...] = (
        jax.lax.dot_general(q_ref[...], wq_ref[...], dn,
                            preferred_element_type=jnp.float32)
        + bq_ref[...]
    )
    ok_ref[...] = (
        jax.lax.dot_general(k_ref[...], wk_ref[...], dn,
                            preferred_element_type=jnp.float32)
        + bk_ref[...]
    )


def _project_qk(q2d, k2d, Wq, bq, Wk, bk):
    return pl.pallas_call(
        _proj_kernel,
        grid=(S // SBLK,),
        in_specs=[
            pl.BlockSpec((SBLK, D), lambda i: (i, 0)),
            pl.BlockSpec((SBLK, D), lambda i: (i, 0)),
            pl.BlockSpec((D, D), lambda i: (0, 0)),
            pl.BlockSpec((D, D), lambda i: (0, 0)),
            pl.BlockSpec((1, D), lambda i: (0, 0)),
            pl.BlockSpec((1, D), lambda i: (0, 0)),
        ],
        out_specs=[
            pl.BlockSpec((SBLK, D), lambda i: (i, 0)),
            pl.BlockSpec((SBLK, D), lambda i: (i, 0)),
        ],
        out_shape=[
            jax.ShapeDtypeStruct((S, D), jnp.float32),
            jax.ShapeDtypeStruct((S, D), jnp.float32),
        ],
    )(q2d, k2d, Wq, Wk, bq.reshape(1, D), bk.reshape(1, D))


def _imp_kernel(qh_ref, kh_ref, imp_ref):
    k = kh_ref[...]
    dn = (((1,), (1,)), ((), ()))
    for j in range(S // QBLK):
        qblk = qh_ref[j * QBLK:(j + 1) * QBLK, :]
        s = jax.lax.dot_general(qblk, k, dn,
                                preferred_element_type=jnp.float32)
        s = s * INV_SQRT_D
        imp = jnp.max(s, axis=1) - jnp.mean(s, axis=1)
        imp_ref[0, 0, j * QBLK:(j + 1) * QBLK] = imp


def _importance(Q2d, K2d):
    out = pl.pallas_call(
        _imp_kernel,
        grid=(H,),
        in_specs=[
            pl.BlockSpec((S, HD), lambda h: (0, h)),
            pl.BlockSpec((S, HD), lambda h: (0, h)),
        ],
        out_specs=pl.BlockSpec((1, 1, S), lambda h: (h, 0, 0)),
        out_shape=jax.ShapeDtypeStruct((H, 1, S), jnp.float32),
    )(Q2d, K2d)
    return out.reshape(H, S)


L = 16
NVEC = S // L
NGRP = NVEC // L
UPW = 128
_NEG = float("-inf")
_BIG = 1 << 20


def _perm(x, sh):
    idx = lax.iota(jnp.int32, L) ^ sh
    return x.at[idx].get(mode="promise_in_bounds")


def _bmax(x):
    for sh in (1, 2, 4, 8):
        x = jnp.maximum(x, _perm(x, sh))
    return x


def _bmin(x):
    for sh in (1, 2, 4, 8):
        x = jnp.minimum(x, _perm(x, sh))
    return x


def _topk_sc_body(imp_hbm, idx_hbm, imp_v, idx_v):
    cid = lax.axis_index("c")
    sid = lax.axis_index("s")
    wid = sid * 2 + cid

    @pl.when(wid < H)
    def _():
        pltpu.sync_copy(imp_hbm.at[wid], imp_v)
        lanes = lax.iota(jnp.int32, L)
        negv = jnp.full((L,), _NEG, jnp.float32)
        bigv = jnp.full((L,), _BIG, jnp.int32)

        def setup_q(q):
            def b(t, acc):
                t2 = t * 2
                v0 = imp_v[pl.ds((q * L + t2) * L, L)]
                v1 = imp_v[pl.ds((q * L + t2 + 1) * L, L)]
                acc = jnp.where(lanes == t2, _bmax(v0), acc)
                return jnp.where(lanes == t2 + 1, _bmax(v1), acc)
            return lax.fori_loop(0, L // 2, b, negv)

        groups = [setup_q(q) for q in range(NGRP)]

        def round_body(u, carry):
            g = list(carry[:NGRP])
            acc = list(carry[NGRP:])
            m = g[0]
            for q in range(1, NGRP):
                m = jnp.maximum(m, g[q])
            gmv = _bmax(m)
            cand = bigv
            for q in range(NGRP):
                cand = jnp.minimum(
                    cand, jnp.where(g[q] == gmv, q * L + lanes, bigv))
            cs = _bmin(cand)[0]
            v = imp_v[pl.ds(cs * L, L)]
            ls = _bmin(jnp.where(v == gmv, lanes, bigv))[0]
            gidx = cs * L + ls
            for a in range(3):
                acc[a] = jnp.where(lanes == (u - a * L), gidx, acc[a])
            nv = jnp.where(lanes == ls, _NEG, v)
            imp_v[pl.ds(cs * L, L)] = nv
            nm = _bmax(nv)
            qs = cs >> 4
            ts = cs & (L - 1)
            qv = (lanes * 0) + qs
            for q in range(NGRP):
                g[q] = jnp.where((lanes + (qv - q) * 64) == ts, nm, g[q])
            return (*g, *acc)

        init = (*groups,
                jnp.full((L,), -1, jnp.int32),
                jnp.full((L,), -1, jnp.int32),
                jnp.full((L,), -1, jnp.int32))
        fin = lax.fori_loop(0, U, round_body, init)
        accs = fin[NGRP:]
        for a in range(UPW // L):
            idx_v[pl.ds(a * L, L)] = (accs[a] if a < 3 else
                                      jnp.full((L,), -1, jnp.int32))
        pltpu.sync_copy(idx_v, idx_hbm.at[wid])


def _topk(imp):
    f = pl.kernel(
        _topk_sc_body,
        out_type=jax.ShapeDtypeStruct((H, UPW), jnp.int32),
        mesh=plsc.VectorSubcoreMesh(core_axis_name="c", subcore_axis_name="s"),
        scratch_types=[
            pltpu.VMEM((S,), jnp.float32),
            pltpu.VMEM((UPW,), jnp.int32),
        ],
    )
    return f(imp)[:, :UP]


def _selattn_kernel(idx_sref, qh_ref, kh_ref, idx_ref, v_ref, wv_ref,
                    bv_ref, wo_ref, bo_ref, out_ref, vmean_ref, base_ref):
    h = pl.program_id(0)

    @pl.when(h == 0)
    def _():
        vmean_ref[...] = jnp.sum(v_ref[...], axis=0, keepdims=True) * (1.0 / S)
        base_ref[...] = bo_ref[...]
        out_ref[...] = jnp.zeros((S, D), jnp.float32)

    dn = (((1,), (1,)), ((), ()))
    q = qh_ref[...]
    k = kh_ref[...]
    idx = idx_ref[0]
    oh = (jax.lax.broadcasted_iota(jnp.int32, (UP, S), 1) == idx)
    oh = oh.astype(jnp.float32)
    qsel = jnp.dot(oh, q, preferred_element_type=jnp.float32)
    s = jax.lax.dot_general(qsel, k, dn,
                            preferred_element_type=jnp.float32) * INV_SQRT_D
    m = jnp.max(s, axis=1, keepdims=True)
    e = jnp.exp(s - m)
    denom = jnp.sum(e, axis=1, keepdims=True) + 1e-8
    w = e / denom
    wv = jnp.dot(w, v_ref[...], preferred_element_type=jnp.float32)
    osel = jax.lax.dot_general(wv, wv_ref[...], dn,
                               preferred_element_type=jnp.float32)
    wsum = jnp.sum(w, axis=1, keepdims=True)
    osel = osel + wsum * bv_ref[0]
    dflt = (jax.lax.dot_general(vmean_ref[...], wv_ref[...], dn,
                                preferred_element_type=jnp.float32)
            + bv_ref[0])
    base_ref[...] += jax.lax.dot_general(dflt, wo_ref[...], dn,
                                         preferred_element_type=jnp.float32)
    corr_out = jax.lax.dot_general(osel - dflt, wo_ref[...], dn,
                                   preferred_element_type=jnp.float32)
    for i in range(U):
        r = idx_sref[h * UP + i]
        out_ref[pl.ds(r, 1), :] += corr_out[i:i + 1, :]

    @pl.when(h == H - 1)
    def _():
        out_ref[...] += base_ref[...]


def _selected_attention(Q2d, K2d, idx, v2d, Wv, bv, Wo, bo):
    idx3 = idx.reshape(H, UP, 1)
    idx_flat = idx.reshape(H * UP)
    bv3 = bv.reshape(H, 1, HD)
    grid_spec = pltpu.PrefetchScalarGridSpec(
        num_scalar_prefetch=1,
        grid=(H,),
        in_specs=[
            pl.BlockSpec((S, HD), lambda h, sref: (0, h)),
            pl.BlockSpec((S, HD), lambda h, sref: (0, h)),
            pl.BlockSpec((1, UP, 1), lambda h, sref: (h, 0, 0)),
            pl.BlockSpec((S, D), lambda h, sref: (0, 0)),
            pl.BlockSpec((HD, D), lambda h, sref: (h, 0)),
            pl.BlockSpec((1, 1, HD), lambda h, sref: (h, 0, 0)),
            pl.BlockSpec((D, HD), lambda h, sref: (0, h)),
            pl.BlockSpec((1, D), lambda h, sref: (0, 0)),
        ],
        out_specs=pl.BlockSpec((S, D), lambda h, sref: (0, 0)),
        scratch_shapes=[
            pltpu.VMEM((1, D), jnp.float32),
            pltpu.VMEM((1, D), jnp.float32),
        ],
    )
    return pl.pallas_call(
        _selattn_kernel,
        grid_spec=grid_spec,
        out_shape=jax.ShapeDtypeStruct((S, D), jnp.float32),
        compiler_params=pltpu.CompilerParams(
            dimension_semantics=("arbitrary",)),
    )(idx_flat, Q2d, K2d, idx3, v2d, Wv, bv3, Wo, bo.reshape(1, D))


@jax.jit
def kernel(queries, keys, values, Wq, bq, Wk, bk, Wv, bv, Wo, bo):
    q2d = queries.reshape(S, D)
    k2d = keys.reshape(S, D)
    v2d = values.reshape(S, D)
    Q2d, K2d = _project_qk(q2d, k2d, Wq, bq, Wk, bk)
    imp = _importance(Q2d, K2d)
    idx = _topk(imp)
    out = _selected_attention(Q2d, K2d, idx, v2d, Wv, bv, Wo, bo)
    return out.reshape(B, S, D)


# speedup vs baseline: 1.1143x; 1.0473x over previous
import functools
import math

import jax
import jax.numpy as jnp
from jax import lax
from jax.experimental import pallas as pl
from jax.experimental.pallas import tpu as pltpu
from jax.experimental.pallas import tpu_sc as plsc

B = 1
S = 2048
D = 2048
H = 16
HD = D // H
U = 38
UP = 40
INV_SQRT_D = 1.0 / math.sqrt(HD)
QBLK = 512
SBLK = 256


def _proj_kernel(q_ref, k_ref, wq_ref, wk_ref, bq_ref, bk_ref,
                 oq_ref, ok_ref):
    dn = (((1,), (1,)), ((), ()))
    oq_ref[...] = (
        jax.lax.dot_general(q_ref[...], wq_ref[...], dn,
                            preferred_element_type=jnp.float32)
        + bq_ref[...]
    )
    ok_ref[...] = (
        jax.lax.dot_general(k_ref[...], wk_ref[...], dn,
                            preferred_element_type=jnp.float32)
        + bk_ref[...]
    )


def _project_qk(q2d, k2d, Wq, bq, Wk, bk):
    return pl.pallas_call(
        _proj_kernel,
        grid=(S // SBLK,),
        in_specs=[
            pl.BlockSpec((SBLK, D), lambda i: (i, 0)),
            pl.BlockSpec((SBLK, D), lambda i: (i, 0)),
            pl.BlockSpec((D, D), lambda i: (0, 0)),
            pl.BlockSpec((D, D), lambda i: (0, 0)),
            pl.BlockSpec((1, D), lambda i: (0, 0)),
            pl.BlockSpec((1, D), lambda i: (0, 0)),
        ],
        out_specs=[
            pl.BlockSpec((SBLK, D), lambda i: (i, 0)),
            pl.BlockSpec((SBLK, D), lambda i: (i, 0)),
        ],
        out_shape=[
            jax.ShapeDtypeStruct((S, D), jnp.float32),
            jax.ShapeDtypeStruct((S, D), jnp.float32),
        ],
    )(q2d, k2d, Wq, Wk, bq.reshape(1, D), bk.reshape(1, D))


def _imp_kernel(qh_ref, kh_ref, imp_ref):
    k = kh_ref[...]
    dn = (((1,), (1,)), ((), ()))
    ksum = jnp.sum(k, axis=0, keepdims=True)
    for j in range(S // QBLK):
        qblk = qh_ref[j * QBLK:(j + 1) * QBLK, :]
        sraw = jax.lax.dot_general(qblk, k, dn,
                                   preferred_element_type=jnp.float32)
        mx = jnp.max(sraw, axis=1) * INV_SQRT_D
        mean = jax.lax.dot_general(qblk, ksum, dn,
                                   preferred_element_type=jnp.float32)
        imp = mx - mean[:, 0] * (INV_SQRT_D / S)
        imp_ref[0, 0, j * QBLK:(j + 1) * QBLK] = imp


def _importance(Q2d, K2d):
    out = pl.pallas_call(
        _imp_kernel,
        grid=(H,),
        in_specs=[
            pl.BlockSpec((S, HD), lambda h: (0, h)),
            pl.BlockSpec((S, HD), lambda h: (0, h)),
        ],
        out_specs=pl.BlockSpec((1, 1, S), lambda h: (h, 0, 0)),
        out_shape=jax.ShapeDtypeStruct((H, 1, S), jnp.float32),
    )(Q2d, K2d)
    return out.reshape(H, S)


L = 16
NVEC = S // L
NGRP = NVEC // L
UPW = 128
_NEG = float("-inf")
_BIG = 1 << 20


def _perm(x, sh):
    idx = lax.iota(jnp.int32, L) ^ sh
    return x.at[idx].get(mode="promise_in_bounds")


def _bmax(x):
    for sh in (1, 2, 4, 8):
        x = jnp.maximum(x, _perm(x, sh))
    return x


def _bmin(x):
    for sh in (1, 2, 4, 8):
        x = jnp.minimum(x, _perm(x, sh))
    return x


def _topk_sc_body(imp_hbm, idx_hbm, imp_v, idx_v):
    cid = lax.axis_index("c")
    sid = lax.axis_index("s")
    wid = sid * 2 + cid

    @pl.when(wid < H)
    def _():
        pltpu.sync_copy(imp_hbm.at[wid], imp_v)
        lanes = lax.iota(jnp.int32, L)
        negv = jnp.full((L,), _NEG, jnp.float32)
        bigv = jnp.full((L,), _BIG, jnp.int32)

        def setup_q(q):
            def b(t, acc):
                t2 = t * 2
                v0 = imp_v[pl.ds((q * L + t2) * L, L)]
                v1 = imp_v[pl.ds((q * L + t2 + 1) * L, L)]
                acc = jnp.where(lanes == t2, _bmax(v0), acc)
                return jnp.where(lanes == t2 + 1, _bmax(v1), acc)
            return lax.fori_loop(0, L // 2, b, negv)

        groups = [setup_q(q) for q in range(NGRP)]

        def round_body(u, carry):
            g = list(carry[:NGRP])
            acc = list(carry[NGRP:])
            m = g[0]
            for q in range(1, NGRP):
                m = jnp.maximum(m, g[q])
            gmv = _bmax(m)
            cand = bigv
            for q in range(NGRP):
                cand = jnp.minimum(
                    cand, jnp.where(g[q] == gmv, q * L + lanes, bigv))
            cs = _bmin(cand)[0]
            v = imp_v[pl.ds(cs * L, L)]
            ls = _bmin(jnp.where(v == gmv, lanes, bigv))[0]
            gidx = cs * L + ls
            for a in range(3):
                acc[a] = jnp.where(lanes == (u - a * L), gidx, acc[a])
            nv = jnp.where(lanes == ls, _NEG, v)
            imp_v[pl.ds(cs * L, L)] = nv
            nm = _bmax(nv)
            qs = cs >> 4
            ts = cs & (L - 1)
            qv = (lanes * 0) + qs
            for q in range(NGRP):
                g[q] = jnp.where((lanes + (qv - q) * 64) == ts, nm, g[q])
            return (*g, *acc)

        init = (*groups,
                jnp.full((L,), -1, jnp.int32),
                jnp.full((L,), -1, jnp.int32),
                jnp.full((L,), -1, jnp.int32))
        fin = lax.fori_loop(0, U, round_body, init)
        accs = fin[NGRP:]
        for a in range(UPW // L):
            idx_v[pl.ds(a * L, L)] = (accs[a] if a < 3 else
                                      jnp.full((L,), -1, jnp.int32))
        pltpu.sync_copy(idx_v, idx_hbm.at[wid])


def _topk(imp):
    f = pl.kernel(
        _topk_sc_body,
        out_type=jax.ShapeDtypeStruct((H, UPW), jnp.int32),
        mesh=plsc.VectorSubcoreMesh(core_axis_name="c", subcore_axis_name="s"),
        scratch_types=[
            pltpu.VMEM((S,), jnp.float32),
            pltpu.VMEM((UPW,), jnp.int32),
        ],
    )
    return f(imp)[:, :UP]


def _selattn_kernel(idx_sref, qh_ref, kh_ref, idx_ref, v_ref, wv_ref,
                    bv_ref, wo_ref, bo_ref, out_ref, vmean_ref, base_ref):
    h = pl.program_id(0)

    @pl.when(h == 0)
    def _():
        vmean_ref[...] = jnp.sum(v_ref[...], axis=0, keepdims=True) * (1.0 / S)
        base_ref[...] = bo_ref[...]
        out_ref[...] = jnp.zeros((S, D), jnp.float32)

    dn = (((1,), (1,)), ((), ()))
    q = qh_ref[...]
    k = kh_ref[...]
    idx = idx_ref[0]
    oh = (jax.lax.broadcasted_iota(jnp.int32, (UP, S), 1) == idx)
    oh = oh.astype(jnp.float32)
    qsel = jnp.dot(oh, q, preferred_element_type=jnp.float32)
    s = jax.lax.dot_general(qsel, k, dn,
                            preferred_element_type=jnp.float32) * INV_SQRT_D
    m = jnp.max(s, axis=1, keepdims=True)
    e = jnp.exp(s - m)
    denom = jnp.sum(e, axis=1, keepdims=True) + 1e-8
    w = e / denom
    wv = jnp.dot(w, v_ref[...], preferred_element_type=jnp.float32)
    osel = jax.lax.dot_general(wv, wv_ref[...], dn,
                               preferred_element_type=jnp.float32)
    wsum = jnp.sum(w, axis=1, keepdims=True)
    osel = osel + wsum * bv_ref[0]
    dflt = (jax.lax.dot_general(vmean_ref[...], wv_ref[...], dn,
                                preferred_element_type=jnp.float32)
            + bv_ref[0])
    base_ref[...] += jax.lax.dot_general(dflt, wo_ref[...], dn,
                                         preferred_element_type=jnp.float32)
    corr_out = jax.lax.dot_general(osel - dflt, wo_ref[...], dn,
                                   preferred_element_type=jnp.float32)
    for i in range(U):
        r = idx_sref[h * UP + i]
        out_ref[pl.ds(r, 1), :] += corr_out[i:i + 1, :]

    @pl.when(h == H - 1)
    def _():
        out_ref[...] += base_ref[...]


def _selected_attention(Q2d, K2d, idx, v2d, Wv, bv, Wo, bo):
    idx3 = idx.reshape(H, UP, 1)
    idx_flat = idx.reshape(H * UP)
    bv3 = bv.reshape(H, 1, HD)
    grid_spec = pltpu.PrefetchScalarGridSpec(
        num_scalar_prefetch=1,
        grid=(H,),
        in_specs=[
            pl.BlockSpec((S, HD), lambda h, sref: (0, h)),
            pl.BlockSpec((S, HD), lambda h, sref: (0, h)),
            pl.BlockSpec((1, UP, 1), lambda h, sref: (h, 0, 0)),
            pl.BlockSpec((S, D), lambda h, sref: (0, 0)),
            pl.BlockSpec((HD, D), lambda h, sref: (h, 0)),
            pl.BlockSpec((1, 1, HD), lambda h, sref: (h, 0, 0)),
            pl.BlockSpec((D, HD), lambda h, sref: (0, h)),
            pl.BlockSpec((1, D), lambda h, sref: (0, 0)),
        ],
        out_specs=pl.BlockSpec((S, D), lambda h, sref: (0, 0)),
        scratch_shapes=[
            pltpu.VMEM((1, D), jnp.float32),
            pltpu.VMEM((1, D), jnp.float32),
        ],
    )
    return pl.pallas_call(
        _selattn_kernel,
        grid_spec=grid_spec,
        out_shape=jax.ShapeDtypeStruct((S, D), jnp.float32),
        compiler_params=pltpu.CompilerParams(
            dimension_semantics=("arbitrary",)),
    )(idx_flat, Q2d, K2d, idx3, v2d, Wv, bv3, Wo, bo.reshape(1, D))


@jax.jit
def kernel(queries, keys, values, Wq, bq, Wk, bk, Wv, bv, Wo, bo):
    q2d = queries.reshape(S, D)
    k2d = keys.reshape(S, D)
    v2d = values.reshape(S, D)
    Q2d, K2d = _project_qk(q2d, k2d, Wq, bq, Wk, bk)
    imp = _importance(Q2d, K2d)
    idx = _topk(imp)
    out = _selected_attention(Q2d, K2d, idx, v2d, Wv, bv, Wo, bo)
    return out.reshape(B, S, D)
